# baseline (device time: 43335 ns/iter reference)
import jax
import jax.numpy as jnp
from jax import lax
from jax.experimental import pallas as pl
from jax.experimental.pallas import tpu as pltpu

N_DEV = 4


def kernel(x, w_mat, scale_x, scale_w):
    m_glob, k_per = x.shape
    k_glob, n = w_mat.shape
    m_per = m_glob // N_DEV

    xseq = (1, 3, 2, 0)
    wseq = (0, 1, 3, 2)

    def body(x_hbm, w_hbm, sx_ref, sw_ref, out_ref,
             x8_ref, recv_ref, xstage, wstage,
             xsems, wsems, send_sems, recv_sems):
        my = lax.axis_index("i")

        def xcopy(i):
            dst = (my + xseq[i]) % N_DEV
            return pltpu.make_async_copy(
                x_hbm.at[pl.ds(dst * m_per, m_per)],
                xstage.at[i],
                xsems.at[i],
            )

        for i in range(N_DEV):
            xcopy(i).start()

        barrier = pltpu.get_barrier_semaphore()
        for off in (1, 2, 3):
            pl.semaphore_signal(
                barrier, inc=1,
                device_id=((my + off) % N_DEV,),
                device_id_type=pl.DeviceIdType.MESH,
            )
        pl.semaphore_wait(barrier, N_DEV - 1)

        rdmas = {}
        for i, off in enumerate(xseq):
            xcopy(i).wait()
            slot = (off - 1) if off else 3
            x8_ref[slot] = xstage[i].astype(jnp.float8_e5m2)
            if off:
                rdma = pltpu.make_async_remote_copy(
                    src_ref=x8_ref.at[slot],
                    dst_ref=recv_ref.at[slot],
                    send_sem=send_sems.at[slot],
                    recv_sem=recv_sems.at[slot],
                    device_id=((my + off) % N_DEV,),
                    device_id_type=pl.DeviceIdType.MESH,
                )
                rdma.start()
                rdmas[off] = rdma

        def wcopy(i):
            src = (my - wseq[i]) % N_DEV
            return pltpu.make_async_copy(
                w_hbm.at[pl.ds(src * k_per, k_per)],
                wstage.at[i % 2],
                wsems.at[i % 2],
            )

        scale = sx_ref[0] * sw_ref[0]

        wcopy(0).start()
        for i, off in enumerate(wseq):
            if i + 1 < N_DEV:
                wcopy(i + 1).start()
            wcopy(i).wait()
            w8 = wstage[i % 2].astype(jnp.float8_e5m2)
            if off:
                rdmas[off].wait_recv()
                a = recv_ref[off - 1]
            else:
                a = x8_ref[3]
            contrib = jnp.dot(a, w8, preferred_element_type=jnp.float32)
            if i == 0:
                out_ref[:, :] = contrib
            elif i < N_DEV - 1:
                out_ref[:, :] = out_ref[:, :] + contrib
            else:
                out_ref[:, :] = jnp.maximum(
                    (out_ref[:, :] + contrib) * scale, 0.0
                )

        for off in (1, 2, 3):
            rdmas[off].wait_send()

    return pl.pallas_call(
        body,
        out_shape=jax.ShapeDtypeStruct((m_per, n), jnp.float32),
        in_specs=[
            pl.BlockSpec(memory_space=pltpu.MemorySpace.HBM),
            pl.BlockSpec(memory_space=pltpu.MemorySpace.HBM),
            pl.BlockSpec(memory_space=pltpu.SMEM),
            pl.BlockSpec(memory_space=pltpu.SMEM),
        ],
        out_specs=pl.BlockSpec(memory_space=pltpu.VMEM),
        scratch_shapes=[
            pltpu.VMEM((N_DEV, m_per, k_per), jnp.float8_e5m2),
            pltpu.VMEM((N_DEV - 1, m_per, k_per), jnp.float8_e5m2),
            pltpu.VMEM((N_DEV, m_per, k_per), jnp.float32),
            pltpu.VMEM((2, k_per, n), jnp.float32),
            pltpu.SemaphoreType.DMA((N_DEV,)),
            pltpu.SemaphoreType.DMA((2,)),
            pltpu.SemaphoreType.DMA((N_DEV - 1,)),
            pltpu.SemaphoreType.DMA((N_DEV - 1,)),
        ],
        compiler_params=pltpu.CompilerParams(
            collective_id=0,
            vmem_limit_bytes=56 * 1024 * 1024,
        ),
    )(x, w_mat, scale_x, scale_w)


# device time: 40710 ns/iter; 1.0645x vs baseline; 1.0645x over previous
import jax
import jax.numpy as jnp
from jax import lax
from jax.experimental import pallas as pl
from jax.experimental.pallas import tpu as pltpu

N_DEV = 4


def kernel(x, w_mat, scale_x, scale_w):
    m_glob, k_per = x.shape
    k_glob, n = w_mat.shape
    m_per = m_glob // N_DEV
    m_half = m_per // 2
    k_half = k_per // 2

    xpieces = ((1, 0), (3, 0), (2, 0), (0, 0), (1, 1), (3, 1), (2, 1), (0, 1))
    wseq = (0, 1, 3, 2)

    def body(x_hbm, w_hbm, sx_ref, sw_ref, out_ref,
             x8_ref, recv_ref, xstage, w8_ref, wstage,
             xsems, wsems, send_sems, recv_sems):
        my = lax.axis_index("i")

        def xcopy(i):
            off, h = xpieces[i]
            dst = (my + off) % N_DEV
            return pltpu.make_async_copy(
                x_hbm.at[pl.ds(dst * m_per + h * m_half, m_half)],
                xstage.at[i],
                xsems.at[i],
            )

        for i in range(len(xpieces)):
            xcopy(i).start()

        barrier = pltpu.get_barrier_semaphore()
        for off in (1, 2, 3):
            pl.semaphore_signal(
                barrier, inc=1,
                device_id=((my + off) % N_DEV,),
                device_id_type=pl.DeviceIdType.MESH,
            )
        pl.semaphore_wait(barrier, N_DEV - 1)

        rdmas = {}
        for i, (off, h) in enumerate(xpieces):
            xcopy(i).wait()
            pidx = (off - 1) * 2 + h if off else 6 + h
            x8_ref[pidx] = xstage[i].astype(jnp.float8_e5m2)
            if off:
                rdma = pltpu.make_async_remote_copy(
                    src_ref=x8_ref.at[pidx],
                    dst_ref=recv_ref.at[pidx],
                    send_sem=send_sems.at[pidx],
                    recv_sem=recv_sems.at[pidx],
                    device_id=((my + off) % N_DEV,),
                    device_id_type=pl.DeviceIdType.MESH,
                )
                rdma.start()
                rdmas[(off, h)] = rdma

        def wcopy(i):
            j, hw = divmod(i, 2)
            src = (my - wseq[j]) % N_DEV
            return pltpu.make_async_copy(
                w_hbm.at[pl.ds(src * k_per + hw * k_half, k_half)],
                wstage.at[i % 2],
                wsems.at[i % 2],
            )

        wcopy(0).start()
        for i in range(2 * N_DEV):
            if i + 1 < 2 * N_DEV:
                wcopy(i + 1).start()
            wcopy(i).wait()
            j, hw = divmod(i, 2)
            w8_ref[j, pl.ds(hw * k_half, k_half), :] = (
                wstage[i % 2].astype(jnp.float8_e5m2)
            )

        scale = sx_ref[0] * sw_ref[0]

        for h in (0, 1):
            row = pl.ds(h * m_half, m_half)
            for j, off in enumerate(wseq):
                if off:
                    rdmas[(off, h)].wait_recv()
                    a = recv_ref[(off - 1) * 2 + h]
                else:
                    a = x8_ref[6 + h]
                contrib = jnp.dot(a, w8_ref[j],
                                  preferred_element_type=jnp.float32)
                if j == 0:
                    out_ref[row, :] = contrib
                elif j < N_DEV - 1:
                    out_ref[row, :] = out_ref[row, :] + contrib
                else:
                    out_ref[row, :] = jnp.maximum(
                        (out_ref[row, :] + contrib) * scale, 0.0
                    )

        for rdma in rdmas.values():
            rdma.wait_send()

    return pl.pallas_call(
        body,
        out_shape=jax.ShapeDtypeStruct((m_per, n), jnp.float32),
        in_specs=[
            pl.BlockSpec(memory_space=pltpu.MemorySpace.HBM),
            pl.BlockSpec(memory_space=pltpu.MemorySpace.HBM),
            pl.BlockSpec(memory_space=pltpu.SMEM),
            pl.BlockSpec(memory_space=pltpu.SMEM),
        ],
        out_specs=pl.BlockSpec(memory_space=pltpu.VMEM),
        scratch_shapes=[
            pltpu.VMEM((8, m_half, k_per), jnp.float8_e5m2),
            pltpu.VMEM((6, m_half, k_per), jnp.float8_e5m2),
            pltpu.VMEM((8, m_half, k_per), jnp.float32),
            pltpu.VMEM((N_DEV, k_per, n), jnp.float8_e5m2),
            pltpu.VMEM((2, k_half, n), jnp.float32),
            pltpu.SemaphoreType.DMA((8,)),
            pltpu.SemaphoreType.DMA((2,)),
            pltpu.SemaphoreType.DMA((6,)),
            pltpu.SemaphoreType.DMA((6,)),
        ],
        compiler_params=pltpu.CompilerParams(
            collective_id=0,
            vmem_limit_bytes=56 * 1024 * 1024,
        ),
    )(x, w_mat, scale_x, scale_w)


# device time: 39527 ns/iter; 1.0963x vs baseline; 1.0299x over previous
import jax
import jax.numpy as jnp
from jax import lax
from jax.experimental import pallas as pl
from jax.experimental.pallas import tpu as pltpu

N_DEV = 4


def kernel(x, w_mat, scale_x, scale_w):
    m_glob, k_per = x.shape
    k_glob, n = w_mat.shape
    m_per = m_glob // N_DEV
    m_half = m_per // 2
    k_half = k_per // 2

    xpieces = ((1, 0), (3, 0), (2, 0), (0, 0), (1, 1), (3, 1), (2, 1), (0, 1))
    wseq = (0, 1, 3, 2)

    def body(x_hbm, w_hbm, sx_ref, sw_ref, out_hbm,
             x8_ref, recv_ref, xstage, w8_ref, wstage, outv_ref,
             xsems, wsems, send_sems, recv_sems, osems):
        my = lax.axis_index("i")

        def xcopy(i):
            off, h = xpieces[i]
            dst = (my + off) % N_DEV
            return pltpu.make_async_copy(
                x_hbm.at[pl.ds(dst * m_per + h * m_half, m_half)],
                xstage.at[i],
                xsems.at[i],
            )

        for i in range(len(xpieces)):
            xcopy(i).start()

        barrier = pltpu.get_barrier_semaphore()
        for off in (1, 2, 3):
            pl.semaphore_signal(
                barrier, inc=1,
                device_id=((my + off) % N_DEV,),
                device_id_type=pl.DeviceIdType.MESH,
            )
        pl.semaphore_wait(barrier, N_DEV - 1)

        rdmas = {}
        for i, (off, h) in enumerate(xpieces):
            xcopy(i).wait()
            pidx = (off - 1) * 2 + h if off else 6 + h
            x8_ref[pidx] = xstage[i].astype(jnp.float8_e5m2)
            if off:
                rdma = pltpu.make_async_remote_copy(
                    src_ref=x8_ref.at[pidx],
                    dst_ref=recv_ref.at[pidx],
                    send_sem=send_sems.at[pidx],
                    recv_sem=recv_sems.at[pidx],
                    device_id=((my + off) % N_DEV,),
                    device_id_type=pl.DeviceIdType.MESH,
                )
                rdma.start()
                rdmas[(off, h)] = rdma

        def wcopy(i):
            j, hw = divmod(i, 2)
            src = (my - wseq[j]) % N_DEV
            return pltpu.make_async_copy(
                w_hbm.at[pl.ds(src * k_per + hw * k_half, k_half)],
                wstage.at[i % 2],
                wsems.at[i % 2],
            )

        wcopy(0).start()
        for i in range(2 * N_DEV):
            if i + 1 < 2 * N_DEV:
                wcopy(i + 1).start()
            wcopy(i).wait()
            j, hw = divmod(i, 2)
            w8_ref[j, pl.ds(hw * k_half, k_half), :] = (
                wstage[i % 2].astype(jnp.float8_e5m2)
            )

        scale = sx_ref[0] * sw_ref[0]

        outcopies = []
        for h in (0, 1):
            for j, off in enumerate(wseq):
                if off:
                    rdmas[(off, h)].wait_recv()
                    a = recv_ref[(off - 1) * 2 + h]
                else:
                    a = x8_ref[6 + h]
                contrib = jnp.dot(a, w8_ref[j],
                                  preferred_element_type=jnp.float32)
                if j == 0:
                    outv_ref[h] = contrib
                elif j < N_DEV - 1:
                    outv_ref[h] = outv_ref[h] + contrib
                else:
                    outv_ref[h] = jnp.maximum(
                        (outv_ref[h] + contrib) * scale, 0.0
                    )
            cp = pltpu.make_async_copy(
                outv_ref.at[h],
                out_hbm.at[pl.ds(h * m_half, m_half)],
                osems.at[h],
            )
            cp.start()
            outcopies.append(cp)

        for cp in outcopies:
            cp.wait()
        for rdma in rdmas.values():
            rdma.wait_send()

    return pl.pallas_call(
        body,
        out_shape=jax.ShapeDtypeStruct((m_per, n), jnp.float32),
        in_specs=[
            pl.BlockSpec(memory_space=pltpu.MemorySpace.HBM),
            pl.BlockSpec(memory_space=pltpu.MemorySpace.HBM),
            pl.BlockSpec(memory_space=pltpu.SMEM),
            pl.BlockSpec(memory_space=pltpu.SMEM),
        ],
        out_specs=pl.BlockSpec(memory_space=pltpu.MemorySpace.HBM),
        scratch_shapes=[
            pltpu.VMEM((8, m_half, k_per), jnp.float8_e5m2),
            pltpu.VMEM((6, m_half, k_per), jnp.float8_e5m2),
            pltpu.VMEM((8, m_half, k_per), jnp.float32),
            pltpu.VMEM((N_DEV, k_per, n), jnp.float8_e5m2),
            pltpu.VMEM((2, k_half, n), jnp.float32),
            pltpu.VMEM((2, m_half, n), jnp.float32),
            pltpu.SemaphoreType.DMA((8,)),
            pltpu.SemaphoreType.DMA((2,)),
            pltpu.SemaphoreType.DMA((6,)),
            pltpu.SemaphoreType.DMA((6,)),
            pltpu.SemaphoreType.DMA((2,)),
        ],
        compiler_params=pltpu.CompilerParams(
            collective_id=0,
            vmem_limit_bytes=56 * 1024 * 1024,
        ),
    )(x, w_mat, scale_x, scale_w)


# device time: 37712 ns/iter; 1.1491x vs baseline; 1.0481x over previous
import jax
import jax.numpy as jnp
from jax import lax
from jax.experimental import pallas as pl
from jax.experimental.pallas import tpu as pltpu

N_DEV = 4
PIECES = 4


def kernel(x, w_mat, scale_x, scale_w):
    m_glob, k_per = x.shape
    k_glob, n = w_mat.shape
    m_per = m_glob // N_DEV
    m_piece = m_per // PIECES
    k_half = k_per // 2

    xpieces = tuple(
        (off, h) for h in range(PIECES) for off in (1, 3, 2)
    ) + tuple((0, h) for h in range(PIECES))
    wseq = (0, 1, 3, 2)

    def body(x_hbm, w_hbm, sx_ref, sw_ref, out_hbm,
             x8_ref, recv_ref, xstage, w8_ref, wstage, outv_ref,
             xsems, wsems, send_sems, recv_sems, osems):
        my = lax.axis_index("i")

        def xcopy(i):
            off, h = xpieces[i]
            dst = (my + off) % N_DEV
            return pltpu.make_async_copy(
                x_hbm.at[pl.ds(dst * m_per + h * m_piece, m_piece)],
                xstage.at[i],
                xsems.at[i],
            )

        for i in range(len(xpieces)):
            xcopy(i).start()

        barrier = pltpu.get_barrier_semaphore()
        for off in (1, 2, 3):
            pl.semaphore_signal(
                barrier, inc=1,
                device_id=((my + off) % N_DEV,),
                device_id_type=pl.DeviceIdType.MESH,
            )
        pl.semaphore_wait(barrier, N_DEV - 1)

        rdmas = {}
        for i, (off, h) in enumerate(xpieces):
            xcopy(i).wait()
            pidx = (off - 1) * PIECES + h if off else 3 * PIECES + h
            x8_ref[pidx] = xstage[i].astype(jnp.float8_e5m2)
            if off:
                rdma = pltpu.make_async_remote_copy(
                    src_ref=x8_ref.at[pidx],
                    dst_ref=recv_ref.at[pidx],
                    send_sem=send_sems.at[pidx],
                    recv_sem=recv_sems.at[pidx],
                    device_id=((my + off) % N_DEV,),
                    device_id_type=pl.DeviceIdType.MESH,
                )
                rdma.start()
                rdmas[(off, h)] = rdma

        def wcopy(i):
            j, hw = divmod(i, 2)
            src = (my - wseq[j]) % N_DEV
            return pltpu.make_async_copy(
                w_hbm.at[pl.ds(src * k_per + hw * k_half, k_half)],
                wstage.at[i % 2],
                wsems.at[i % 2],
            )

        wcopy(0).start()
        for i in range(2 * N_DEV):
            if i + 1 < 2 * N_DEV:
                wcopy(i + 1).start()
            wcopy(i).wait()
            j, hw = divmod(i, 2)
            w8_ref[j, pl.ds(hw * k_half, k_half), :] = (
                wstage[i % 2].astype(jnp.float8_e5m2)
            )

        scale = sx_ref[0] * sw_ref[0]

        outcopies = []
        for h in range(PIECES):
            for j, off in enumerate(wseq):
                if off:
                    rdmas[(off, h)].wait_recv()
                    a = recv_ref[(off - 1) * PIECES + h]
                else:
                    a = x8_ref[3 * PIECES + h]
                contrib = jnp.dot(a, w8_ref[j],
                                  preferred_element_type=jnp.float32)
                if j == 0:
                    outv_ref[h] = contrib
                elif j < N_DEV - 1:
                    outv_ref[h] = outv_ref[h] + contrib
                else:
                    outv_ref[h] = jnp.maximum(
                        (outv_ref[h] + contrib) * scale, 0.0
                    )
            cp = pltpu.make_async_copy(
                outv_ref.at[h],
                out_hbm.at[pl.ds(h * m_piece, m_piece)],
                osems.at[h],
            )
            cp.start()
            outcopies.append(cp)

        for cp in outcopies:
            cp.wait()
        for rdma in rdmas.values():
            rdma.wait_send()

    return pl.pallas_call(
        body,
        out_shape=jax.ShapeDtypeStruct((m_per, n), jnp.float32),
        in_specs=[
            pl.BlockSpec(memory_space=pltpu.MemorySpace.HBM),
            pl.BlockSpec(memory_space=pltpu.MemorySpace.HBM),
            pl.BlockSpec(memory_space=pltpu.SMEM),
            pl.BlockSpec(memory_space=pltpu.SMEM),
        ],
        out_specs=pl.BlockSpec(memory_space=pltpu.MemorySpace.HBM),
        scratch_shapes=[
            pltpu.VMEM((4 * PIECES, m_piece, k_per), jnp.float8_e5m2),
            pltpu.VMEM((3 * PIECES, m_piece, k_per), jnp.float8_e5m2),
            pltpu.VMEM((4 * PIECES, m_piece, k_per), jnp.float32),
            pltpu.VMEM((N_DEV, k_per, n), jnp.float8_e5m2),
            pltpu.VMEM((2, k_half, n), jnp.float32),
            pltpu.VMEM((PIECES, m_piece, n), jnp.float32),
            pltpu.SemaphoreType.DMA((4 * PIECES,)),
            pltpu.SemaphoreType.DMA((2,)),
            pltpu.SemaphoreType.DMA((3 * PIECES,)),
            pltpu.SemaphoreType.DMA((3 * PIECES,)),
            pltpu.SemaphoreType.DMA((PIECES,)),
        ],
        compiler_params=pltpu.CompilerParams(
            collective_id=0,
            vmem_limit_bytes=56 * 1024 * 1024,
        ),
    )(x, w_mat, scale_x, scale_w)
